# speedup vs baseline: 1.9367x; 1.9367x over previous
import functools

import jax
import jax.numpy as jnp
from jax import lax
from jax.experimental import pallas as pl
from jax.experimental.pallas import tpu as pltpu
from jax.experimental.pallas import tpu_sc as plsc

VOCAB = 8192
D = 8192
B_TOTAL = 8192
NC = 2
NS = 16
NW = NC * NS
ROWS_PER_W = B_TOTAL // NW
CH = 4
NCHUNK = ROWS_PER_W // CH
NPAIR = NCHUNK // 2


def _gather_body(idx_hbm, table_hbm, out_hbm,
                 idx_v, buf0, buf1, gsem0, gsem1, ssem0, ssem1):
    wid = lax.axis_index("s") * NC + lax.axis_index("c")
    base = wid * ROWS_PER_W

    pltpu.sync_copy(idx_hbm.at[wid], idx_v)

    bufs = (buf0, buf1)
    gsems = (gsem0, gsem1)
    ssems = (ssem0, ssem1)

    def pair(g, carry):
        for b in range(2):
            c = 2 * g + b

            @pl.when(g >= 1)
            def _():
                pltpu.make_async_copy(
                    bufs[b], out_hbm.at[pl.ds(base, CH)], ssems[b]).wait()

            pltpu.async_copy(
                table_hbm.at[idx_v.at[c]], bufs[b], gsems[b]).wait()

            pltpu.async_copy(
                bufs[b], out_hbm.at[pl.ds(base + c * CH, CH)], ssems[b])
        return carry

    lax.fori_loop(0, NPAIR, pair, 0)

    for b in range(2):
        pltpu.make_async_copy(
            bufs[b], out_hbm.at[pl.ds(base, CH)], ssems[b]).wait()


_sc_gather = functools.partial(
    pl.kernel,
    out_type=jax.ShapeDtypeStruct((B_TOTAL, D), jnp.float32),
    mesh=plsc.VectorSubcoreMesh(core_axis_name="c", subcore_axis_name="s"),
    scratch_types=[
        pltpu.VMEM((NCHUNK, CH), jnp.int32),
        pltpu.VMEM((CH, D), jnp.float32),
        pltpu.VMEM((CH, D), jnp.float32),
        pltpu.SemaphoreType.DMA,
        pltpu.SemaphoreType.DMA,
        pltpu.SemaphoreType.DMA,
        pltpu.SemaphoreType.DMA,
    ],
)(_gather_body)


@jax.jit
def kernel(idx, table):
    idx_blocks = idx.reshape(NW, NCHUNK, CH).astype(jnp.int32)
    out = _sc_gather(idx_blocks, table)
    return out.reshape(idx.shape[0], idx.shape[1], D)


# speedup vs baseline: 1.9538x; 1.0088x over previous
import functools

import jax
import jax.numpy as jnp
from jax import lax
from jax.experimental import pallas as pl
from jax.experimental.pallas import tpu as pltpu
from jax.experimental.pallas import tpu_sc as plsc

VOCAB = 8192
D = 8192
B_TOTAL = 8192
NC = 2
NS = 16
NW = NC * NS
ROWS_PER_W = B_TOTAL // NW
CH = 4
NCHUNK = ROWS_PER_W // CH
NBUF = 3


def _gather_body(idx_hbm, table_hbm, out_hbm,
                 idx_v, buf0, buf1, buf2,
                 gsem0, gsem1, gsem2, ssem0, ssem1, ssem2):
    wid = lax.axis_index("s") * NC + lax.axis_index("c")
    base = wid * ROWS_PER_W

    pltpu.sync_copy(idx_hbm.at[wid], idx_v)

    bufs = (buf0, buf1, buf2)
    gsems = (gsem0, gsem1, gsem2)
    ssems = (ssem0, ssem1, ssem2)

    def issue_gather(c, b):
        pltpu.async_copy(table_hbm.at[idx_v.at[c]], bufs[b], gsems[b])

    def wait_gather(c, b):
        pltpu.make_async_copy(
            table_hbm.at[idx_v.at[c]], bufs[b], gsems[b]).wait()

    def issue_scatter(c, b):
        pltpu.async_copy(
            bufs[b], out_hbm.at[pl.ds(base + c * CH, CH)], ssems[b])

    def wait_scatter(b):
        pltpu.make_async_copy(
            bufs[b], out_hbm.at[pl.ds(base, CH)], ssems[b]).wait()

    def step(c, j):
        b2 = (j + 2) % NBUF
        wait_gather(c, j)
        issue_scatter(c, j)

        @pl.when(c + 2 < NCHUNK)
        def _():
            @pl.when(c >= 1)
            def _():
                wait_scatter(b2)
            issue_gather(c + 2, b2)

    issue_gather(0, 0)
    issue_gather(1, 1)

    def group(g, carry):
        for j in range(NBUF):
            step(NBUF * g + j, j)
        return carry

    lax.fori_loop(0, (NCHUNK - 1) // NBUF, group, 0)

    last = NCHUNK - 1
    wait_gather(last, last % NBUF)
    issue_scatter(last, last % NBUF)
    for b in range(NBUF):
        wait_scatter(b)


_sc_gather = functools.partial(
    pl.kernel,
    out_type=jax.ShapeDtypeStruct((B_TOTAL, D), jnp.float32),
    mesh=plsc.VectorSubcoreMesh(core_axis_name="c", subcore_axis_name="s"),
    scratch_types=[
        pltpu.VMEM((NCHUNK, CH), jnp.int32),
        pltpu.VMEM((CH, D), jnp.float32),
        pltpu.VMEM((CH, D), jnp.float32),
        pltpu.VMEM((CH, D), jnp.float32),
        pltpu.SemaphoreType.DMA,
        pltpu.SemaphoreType.DMA,
        pltpu.SemaphoreType.DMA,
        pltpu.SemaphoreType.DMA,
        pltpu.SemaphoreType.DMA,
        pltpu.SemaphoreType.DMA,
    ],
)(_gather_body)


@jax.jit
def kernel(idx, table):
    idx_blocks = idx.reshape(NW, NCHUNK, CH).astype(jnp.int32)
    out = _sc_gather(idx_blocks, table)
    return out.reshape(idx.shape[0], idx.shape[1], D)
